# baseline (speedup 1.0000x reference)
import jax
import jax.numpy as jnp
from jax.experimental import pallas as pl
from jax.experimental.pallas import tpu as pltpu

B, F, D = 4096, 26, 128
ROWS = B * F
RB = 2048
K = 16
N = ROWS // RB


def _body(idx_ref, x_hbm, w_ref, b_ref, o_hbm,
          xs, ys, wp_ref, in_sems, out_sems):
    i = pl.program_id(0)

    def in_copy(c, slot):
        return pltpu.make_async_copy(
            x_hbm.at[pl.ds(c * RB, RB), :],
            xs.at[slot],
            in_sems.at[slot],
        )

    def out_copy(c, slot):
        return pltpu.make_async_copy(
            ys.at[slot],
            o_hbm.at[pl.ds(c * RB, RB), :],
            out_sems.at[slot],
        )

    @pl.when(i == 0)
    def _prologue():
        perm = idx_ref[0, :]
        iota = jax.lax.broadcasted_iota(jnp.int32, (D, D), 0)
        onehot = (iota == perm[None, :]).astype(jnp.float32)
        wp_ref[...] = jax.lax.dot(onehot, w_ref[...],
                                  preferred_element_type=jnp.float32)
        for k in range(K):
            in_copy(k, k).start()

    slot = jax.lax.rem(i, K)
    in_copy(i, slot).wait()

    @pl.when(i >= K)
    def _():
        out_copy(i - K, slot).wait()

    y = jax.lax.dot(xs[slot], wp_ref[...],
                    preferred_element_type=jnp.float32)
    ys[slot] = y + b_ref[0, :][None, :]
    out_copy(i, slot).start()

    @pl.when(i + K < N)
    def _():
        in_copy(i + K, slot).start()

    @pl.when(i == N - 1)
    def _drain():
        for d in range(K):
            c = N - K + d
            if c >= 0:
                out_copy(c, c % K).wait()


@jax.jit
def _run(x2d, idx2d, W, b2d):
    return pl.pallas_call(
        _body,
        grid=(N,),
        in_specs=[
            pl.BlockSpec((1, D), lambda i: (0, 0)),
            pl.BlockSpec(memory_space=pltpu.MemorySpace.HBM),
            pl.BlockSpec((D, D), lambda i: (0, 0)),
            pl.BlockSpec((1, D), lambda i: (0, 0)),
        ],
        out_specs=pl.BlockSpec(memory_space=pltpu.MemorySpace.HBM),
        out_shape=jax.ShapeDtypeStruct((ROWS, D), jnp.float32),
        scratch_shapes=[
            pltpu.VMEM((K, RB, D), jnp.float32),
            pltpu.VMEM((K, RB, D), jnp.float32),
            pltpu.VMEM((D, D), jnp.float32),
            pltpu.SemaphoreType.DMA((K,)),
            pltpu.SemaphoreType.DMA((K,)),
        ],
        compiler_params=pltpu.CompilerParams(
            dimension_semantics=("arbitrary",),
        ),
    )(idx2d, x2d, W, b2d)


def kernel(x, permute_idx, W, b):
    idx2d = permute_idx.astype(jnp.int32).reshape(1, D)
    b2d = b.reshape(1, D)
    x2d = jnp.transpose(x, (1, 0, 2)).reshape(ROWS, D)
    out2d = _run(x2d, idx2d, W, b2d)
    return jnp.transpose(out2d.reshape(F, B, D), (1, 0, 2))


# speedup vs baseline: 1.0116x; 1.0116x over previous
import jax
import jax.numpy as jnp
from jax.experimental import pallas as pl
from jax.experimental.pallas import tpu as pltpu

B, F, D = 4096, 26, 128
ROWS = B * F
RB = 8192
K = 6
N = ROWS // RB


def _body(idx_ref, x_hbm, w_ref, b_ref, o_hbm,
          xs, ys, wp_ref, in_sems, out_sems):
    i = pl.program_id(0)

    def in_copy(c, slot):
        return pltpu.make_async_copy(
            x_hbm.at[pl.ds(c * RB, RB), :],
            xs.at[slot],
            in_sems.at[slot],
        )

    def out_copy(c, slot):
        return pltpu.make_async_copy(
            ys.at[slot],
            o_hbm.at[pl.ds(c * RB, RB), :],
            out_sems.at[slot],
        )

    @pl.when(i == 0)
    def _prologue():
        perm = idx_ref[0, :]
        iota = jax.lax.broadcasted_iota(jnp.int32, (D, D), 0)
        onehot = (iota == perm[None, :]).astype(jnp.float32)
        wp_ref[...] = jax.lax.dot(onehot, w_ref[...],
                                  preferred_element_type=jnp.float32)
        for k in range(K):
            in_copy(k, k).start()

    slot = jax.lax.rem(i, K)
    in_copy(i, slot).wait()

    @pl.when(i >= K)
    def _():
        out_copy(i - K, slot).wait()

    y = jax.lax.dot(xs[slot], wp_ref[...],
                    preferred_element_type=jnp.float32)
    ys[slot] = y + b_ref[0, :][None, :]
    out_copy(i, slot).start()

    @pl.when(i + K < N)
    def _():
        in_copy(i + K, slot).start()

    @pl.when(i == N - 1)
    def _drain():
        for d in range(K):
            c = N - K + d
            if c >= 0:
                out_copy(c, c % K).wait()


@jax.jit
def _run(x2d, idx2d, W, b2d):
    return pl.pallas_call(
        _body,
        grid=(N,),
        in_specs=[
            pl.BlockSpec((1, D), lambda i: (0, 0)),
            pl.BlockSpec(memory_space=pltpu.MemorySpace.HBM),
            pl.BlockSpec((D, D), lambda i: (0, 0)),
            pl.BlockSpec((1, D), lambda i: (0, 0)),
        ],
        out_specs=pl.BlockSpec(memory_space=pltpu.MemorySpace.HBM),
        out_shape=jax.ShapeDtypeStruct((ROWS, D), jnp.float32),
        scratch_shapes=[
            pltpu.VMEM((K, RB, D), jnp.float32),
            pltpu.VMEM((K, RB, D), jnp.float32),
            pltpu.VMEM((D, D), jnp.float32),
            pltpu.SemaphoreType.DMA((K,)),
            pltpu.SemaphoreType.DMA((K,)),
        ],
        compiler_params=pltpu.CompilerParams(
            dimension_semantics=("arbitrary",),
        ),
    )(idx2d, x2d, W, b2d)


def kernel(x, permute_idx, W, b):
    idx2d = permute_idx.astype(jnp.int32).reshape(1, D)
    b2d = b.reshape(1, D)
    x2d = jnp.transpose(x, (1, 0, 2)).reshape(ROWS, D)
    out2d = _run(x2d, idx2d, W, b2d)
    return jnp.transpose(out2d.reshape(F, B, D), (1, 0, 2))


# speedup vs baseline: 1.0189x; 1.0072x over previous
import jax
import jax.numpy as jnp
from jax.experimental import pallas as pl
from jax.experimental.pallas import tpu as pltpu

B, F, D = 4096, 26, 128
ROWS = B * F
RB = 13312
K = 4
N = ROWS // RB


def _body(idx_ref, x_hbm, w_ref, b_ref, o_hbm,
          xs, ys, wp_ref, in_sems, out_sems):
    i = pl.program_id(0)

    def in_copy(c, slot):
        return pltpu.make_async_copy(
            x_hbm.at[pl.ds(c * RB, RB), :],
            xs.at[slot],
            in_sems.at[slot],
        )

    def out_copy(c, slot):
        return pltpu.make_async_copy(
            ys.at[slot],
            o_hbm.at[pl.ds(c * RB, RB), :],
            out_sems.at[slot],
        )

    @pl.when(i == 0)
    def _prologue():
        perm = idx_ref[0, :]
        iota = jax.lax.broadcasted_iota(jnp.int32, (D, D), 0)
        onehot = (iota == perm[None, :]).astype(jnp.float32)
        wp_ref[...] = jax.lax.dot(onehot, w_ref[...],
                                  preferred_element_type=jnp.float32)
        for k in range(K):
            in_copy(k, k).start()

    slot = jax.lax.rem(i, K)
    in_copy(i, slot).wait()

    @pl.when(i >= K)
    def _():
        out_copy(i - K, slot).wait()

    y = jax.lax.dot(xs[slot], wp_ref[...],
                    preferred_element_type=jnp.float32)
    ys[slot] = y + b_ref[0, :][None, :]
    out_copy(i, slot).start()

    @pl.when(i + K < N)
    def _():
        in_copy(i + K, slot).start()

    @pl.when(i == N - 1)
    def _drain():
        for d in range(K):
            c = N - K + d
            if c >= 0:
                out_copy(c, c % K).wait()


@jax.jit
def _run(x2d, idx2d, W, b2d):
    return pl.pallas_call(
        _body,
        grid=(N,),
        in_specs=[
            pl.BlockSpec((1, D), lambda i: (0, 0)),
            pl.BlockSpec(memory_space=pltpu.MemorySpace.HBM),
            pl.BlockSpec((D, D), lambda i: (0, 0)),
            pl.BlockSpec((1, D), lambda i: (0, 0)),
        ],
        out_specs=pl.BlockSpec(memory_space=pltpu.MemorySpace.HBM),
        out_shape=jax.ShapeDtypeStruct((ROWS, D), jnp.float32),
        scratch_shapes=[
            pltpu.VMEM((K, RB, D), jnp.float32),
            pltpu.VMEM((K, RB, D), jnp.float32),
            pltpu.VMEM((D, D), jnp.float32),
            pltpu.SemaphoreType.DMA((K,)),
            pltpu.SemaphoreType.DMA((K,)),
        ],
        compiler_params=pltpu.CompilerParams(
            dimension_semantics=("arbitrary",),
        ),
    )(idx2d, x2d, W, b2d)


def kernel(x, permute_idx, W, b):
    idx2d = permute_idx.astype(jnp.int32).reshape(1, D)
    b2d = b.reshape(1, D)
    x2d = jnp.transpose(x, (1, 0, 2)).reshape(ROWS, D)
    out2d = _run(x2d, idx2d, W, b2d)
    return jnp.transpose(out2d.reshape(F, B, D), (1, 0, 2))
